# baseline (speedup 1.0000x reference)
import jax
import jax.numpy as jnp
from jax import lax
from jax.experimental import pallas as pl
from jax.experimental.pallas import tpu as pltpu
from jax.experimental.pallas import tpu_sc as plsc

_D_IN = 4096
_D_H = 2048
_NC = 2
_NS = 16
_L = 16
_COLS_W = 128
_UC = _COLS_W // _L
_ROWS_W = _D_IN // 2
_RCH = 256
_NRCH = _ROWS_W // _RCH
_RG = 4
_NPAIR = _NC * _NS // 2
_PK = plsc.PackFormat.INTERLEAVED


def _splat(v, i):
    idx = jnp.full((_L,), i, dtype=jnp.int32)
    return v.at[idx].get(mode="promise_in_bounds")


def _round_pair(x, y):
    return plsc.unpack(plsc.pack(x, y, format=_PK), format=_PK)


def _round8(v):
    v = list(v)
    for u in range(0, 8, 2):
        v[u], v[u + 1] = _round_pair(v[u], v[u + 1])
    return v


def _sc_body(w_hbm, a_hbm, b_hbm, ws_hbm, be_hbm, bs_hbm, out_hbm,
             buf0, buf1, a_v, b_v, ws1_v, ws2_v, be_v, bs_v, pv_v,
             acc_v, mate_v, spx, sem0, sem1):
    cid = lax.axis_index("c")
    sid = lax.axis_index("s")
    pair = sid // 2
    half = sid % 2
    col0 = cid * (_NS // 2 * _COLS_W) + pair * _COLS_W
    row0 = half * _ROWS_W
    bufs = [buf0, buf1]
    sems = [sem0, sem1]
    handles = [
        pltpu.async_copy(
            w_hbm.at[pl.ds(row0 + c * _RCH, _RCH), pl.ds(col0, _COLS_W)],
            bufs[c], sems[c])
        for c in range(2)
    ]
    pltpu.sync_copy(a_hbm.at[pl.ds(row0, _ROWS_W)], a_v)
    pltpu.sync_copy(b_hbm.at[pl.ds(row0, _ROWS_W)], b_v)
    pltpu.sync_copy(ws_hbm.at[pl.ds(col0, _COLS_W)], ws1_v)
    pltpu.sync_copy(ws_hbm.at[pl.ds(_D_H + col0, _COLS_W)], ws2_v)
    pltpu.sync_copy(be_hbm.at[pl.ds(col0, _COLS_W)], be_v)

    zero = jnp.zeros((_L,), jnp.float32)
    accs = (zero,) * (2 * _UC)
    for c in range(_NRCH):
        handles[c % 2].wait()
        buf = bufs[c % 2]

        def grp_body(g, carry, buf=buf, c=c):
            a1 = list(carry[:_UC])
            a2 = list(carry[_UC:])
            r0 = g * _RG
            av = a_v[pl.ds(c * _RCH + r0, _L)]
            bv = b_v[pl.ds(c * _RCH + r0, _L)]
            for i in range(_RG):
                ai = _splat(av, i)
                bi = _splat(bv, i)
                w = _round8([buf[r0 + i, pl.ds(u * _L, _L)]
                             for u in range(_UC)])
                for u in range(_UC):
                    a1[u] = a1[u] + ai * w[u]
                    a2[u] = a2[u] + bi * w[u]
            return tuple(a1) + tuple(a2)

        accs = lax.fori_loop(0, _RCH // _RG, grp_body, accs)
        if c + 2 < _NRCH:
            handles[c % 2] = pltpu.async_copy(
                w_hbm.at[pl.ds(row0 + (c + 2) * _RCH, _RCH),
                         pl.ds(col0, _COLS_W)],
                bufs[c % 2], sems[c % 2])

    for k in range(2 * _UC):
        acc_v[pl.ds(k * _L, _L)] = accs[k]
    pltpu.sync_copy(acc_v, spx.at[sid])
    plsc.subcore_barrier()

    @pl.when(half == 0)
    def _():
        pltpu.sync_copy(spx.at[sid + 1], mate_v)
        ea = [acc_v[pl.ds(u * _L, _L)] + mate_v[pl.ds(u * _L, _L)]
              + be_v[pl.ds(u * _L, _L)] for u in range(_UC)]
        eb = [acc_v[pl.ds((_UC + u) * _L, _L)]
              + mate_v[pl.ds((_UC + u) * _L, _L)]
              + be_v[pl.ds(u * _L, _L)] for u in range(_UC)]
        ea = _round8(ea)
        eb = _round8(eb)
        tot = jnp.zeros((_L,), jnp.float32)
        for u in range(_UC):
            tot = tot + ea[u] * ws1_v[pl.ds(u * _L, _L)]
            tot = tot + eb[u] * ws2_v[pl.ds(u * _L, _L)]

        @pl.when((cid == 0) & (sid == 0))
        def _():
            pltpu.sync_copy(bs_hbm, bs_v)
            tot2 = tot + bs_v[...]
            pv_v[...] = tot2

        @pl.when((cid != 0) | (sid != 0))
        def _():
            pv_v[...] = tot

        pltpu.sync_copy(pv_v, out_hbm.at[cid * (_NS // 2) + pair])


def kernel(a, b, W_embed, b_embed, W_scorer, b_scorer):
    f32 = jnp.float32
    bf = jnp.bfloat16
    a_r = a.astype(bf).astype(f32)
    b_r = b.astype(bf).astype(f32)
    ws_r = W_scorer.reshape(-1).astype(bf).astype(f32)
    bs16 = jnp.pad(b_scorer, (0, _L - 1))
    mesh = plsc.VectorSubcoreMesh(core_axis_name="c", subcore_axis_name="s")
    run = pl.kernel(
        _sc_body,
        mesh=mesh,
        compiler_params=pltpu.CompilerParams(needs_layout_passes=False),
        out_type=jax.ShapeDtypeStruct((_NPAIR, _L), jnp.float32),
        scratch_types=[
            pltpu.VMEM((_RCH, _COLS_W), jnp.float32),
            pltpu.VMEM((_RCH, _COLS_W), jnp.float32),
            pltpu.VMEM((_ROWS_W,), jnp.float32),
            pltpu.VMEM((_ROWS_W,), jnp.float32),
            pltpu.VMEM((_COLS_W,), jnp.float32),
            pltpu.VMEM((_COLS_W,), jnp.float32),
            pltpu.VMEM((_COLS_W,), jnp.float32),
            pltpu.VMEM((_L,), jnp.float32),
            pltpu.VMEM((_L,), jnp.float32),
            pltpu.VMEM((2 * _COLS_W,), jnp.float32),
            pltpu.VMEM((2 * _COLS_W,), jnp.float32),
            pltpu.VMEM_SHARED((_NS, 2 * _COLS_W), jnp.float32),
            pltpu.SemaphoreType.DMA,
            pltpu.SemaphoreType.DMA,
        ],
    )
    parts = run(W_embed, a_r, b_r, ws_r, b_embed, bs16)
    return jnp.sum(parts)
